# baseline (speedup 1.0000x reference)
import functools

import jax
import jax.numpy as jnp
from jax import lax
from jax.experimental import pallas as pl
from jax.experimental.pallas import tpu as pltpu
from jax.experimental.pallas import tpu_sc as plsc

LANES = 16
CHUNK = 128


def _build(total_rows, vocab, embed, seq, n_workers):
    per_w = total_rows // n_workers
    n_chunks = per_w // CHUNK
    mesh = plsc.VectorSubcoreMesh(core_axis_name="c", subcore_axis_name="s")
    num_cores = 2

    @functools.partial(
        pl.kernel,
        mesh=mesh,
        compiler_params=pltpu.CompilerParams(use_tc_tiling_on_sc=False),
        out_type=jax.ShapeDtypeStruct((total_rows, embed), jnp.float32),
        scratch_types=[
            pltpu.VMEM((CHUNK,), jnp.int32),
            pltpu.VMEM((CHUNK, embed), jnp.float32),
            pltpu.VMEM((2 * seq, embed), jnp.float32),
            pltpu.SemaphoreType.DMA,
        ],
    )
    def emb_kernel(idx_hbm, table_hbm, pos_hbm, out_hbm,
                   idx_v, rows_v, pos2_v, sem):
        wid = lax.axis_index("s") * num_cores + lax.axis_index("c")
        base0 = wid * per_w

        pltpu.sync_copy(pos_hbm, pos2_v.at[pl.ds(0, seq)])
        pltpu.sync_copy(pos_hbm, pos2_v.at[pl.ds(seq, seq)])

        def chunk_body(c, _):
            base = base0 + c * CHUNK
            pltpu.sync_copy(idx_hbm.at[pl.ds(base, CHUNK)], idx_v)
            pltpu.async_copy(table_hbm.at[idx_v], rows_v, sem).wait()

            p0 = lax.rem(base, seq)

            def add_body(r, _):
                pr = p0 + r
                for j in range(embed // LANES):
                    sl = pl.ds(j * LANES, LANES)
                    rows_v[r, sl] = rows_v[r, sl] + pos2_v[pr, sl]
                return _

            lax.fori_loop(0, CHUNK, add_body, None)
            pltpu.sync_copy(rows_v, out_hbm.at[pl.ds(base, CHUNK)])
            return _

        lax.fori_loop(0, n_chunks, chunk_body, None)

    return emb_kernel


def kernel(inputs, word_table, pos_table):
    batch, seq = inputs.shape
    vocab, embed = word_table.shape
    total_rows = batch * seq
    n_workers = 32

    idx = inputs.reshape(total_rows).astype(jnp.int32)
    fn = _build(total_rows, vocab, embed, seq, n_workers)
    out = fn(idx, word_table, pos_table)
    return out.reshape(batch, seq, embed)


# speedup vs baseline: 1.3245x; 1.3245x over previous
import functools

import jax
import jax.numpy as jnp
from jax import lax
from jax.experimental import pallas as pl
from jax.experimental.pallas import tpu as pltpu
from jax.experimental.pallas import tpu_sc as plsc

LANES = 16
CHUNK = 128
NBUF = 4


def _build(total_rows, vocab, embed, seq, n_workers):
    per_w = total_rows // n_workers
    n_chunks = per_w // CHUNK
    n_groups = n_chunks // NBUF
    mesh = plsc.VectorSubcoreMesh(core_axis_name="c", subcore_axis_name="s")
    num_cores = 2

    @functools.partial(
        pl.kernel,
        mesh=mesh,
        compiler_params=pltpu.CompilerParams(use_tc_tiling_on_sc=False),
        out_type=jax.ShapeDtypeStruct((total_rows, embed), jnp.float32),
        scratch_types=[
            pltpu.VMEM((n_chunks, CHUNK), jnp.int32),
            pltpu.VMEM((NBUF, CHUNK, embed), jnp.float32),
            pltpu.VMEM((2 * seq, embed), jnp.float32),
            pltpu.SemaphoreType.DMA,
            pltpu.SemaphoreType.DMA,
            pltpu.SemaphoreType.DMA,
            pltpu.SemaphoreType.DMA,
            pltpu.SemaphoreType.DMA,
            pltpu.SemaphoreType.DMA,
            pltpu.SemaphoreType.DMA,
            pltpu.SemaphoreType.DMA,
        ],
    )
    def emb_kernel(idx_hbm, table_hbm, pos_hbm, out_hbm,
                   idx_all, rows_v, pos2_v, *sems):
        semg = sems[:NBUF]
        semw = sems[NBUF:]
        wid = lax.axis_index("s") * num_cores + lax.axis_index("c")
        base0 = wid * per_w

        pltpu.sync_copy(pos_hbm, pos2_v.at[pl.ds(0, seq)])
        pltpu.sync_copy(pos_hbm, pos2_v.at[pl.ds(seq, seq)])
        pltpu.sync_copy(idx_hbm.at[pl.ds(wid * n_chunks, n_chunks)], idx_all)

        def fire(c, b):
            pltpu.async_copy(table_hbm.at[idx_all.at[c]], rows_v.at[b],
                             semg[b])

        def drain_g(b):
            pltpu.make_async_copy(table_hbm.at[idx_all.at[0]], rows_v.at[b],
                                  semg[b]).wait()

        def drain_w(b):
            pltpu.make_async_copy(rows_v.at[b], out_hbm.at[pl.ds(0, CHUNK)],
                                  semw[b]).wait()

        def add_pos(c, b):
            p0 = lax.rem(c * CHUNK, seq)

            def add_body(r0, _):
                for rr in range(8):
                    r = r0 * 8 + rr
                    for j in range(embed // LANES):
                        sl = pl.ds(j * LANES, LANES)
                        plsc.addupdate(rows_v.at[b, r, sl],
                                       pos2_v[p0 + r, sl])
                return _

            lax.fori_loop(0, CHUNK // 8, add_body, None)

        def step(c, b, wait_w, fire_ahead):
            drain_g(b)
            add_pos(c, b)
            pltpu.async_copy(rows_v.at[b],
                             out_hbm.at[pl.ds(base0 + c * CHUNK, CHUNK)],
                             semw[b])
            bf = (b + NBUF - 1) % NBUF
            if wait_w:
                drain_w(bf)
            if fire_ahead:
                fire(c + NBUF - 1, bf)

        for b in range(NBUF - 1):
            fire(b, b)
        for b in range(NBUF):
            step(b, b, wait_w=(b > 0), fire_ahead=True)

        def group_body(g, _):
            c0 = g * NBUF
            for b in range(NBUF):
                step(c0 + b, b, wait_w=True, fire_ahead=True)
            return _

        lax.fori_loop(1, n_groups - 1, group_body, None)

        cL = (n_groups - 1) * NBUF
        step(cL, 0, wait_w=True, fire_ahead=True)
        for b in range(1, NBUF):
            step(cL + b, b, wait_w=True, fire_ahead=False)
        drain_w(NBUF - 1)

    return emb_kernel


def kernel(inputs, word_table, pos_table):
    batch, seq = inputs.shape
    vocab, embed = word_table.shape
    total_rows = batch * seq
    n_workers = 32

    idx = inputs.reshape(total_rows // CHUNK, CHUNK).astype(jnp.int32)
    fn = _build(total_rows, vocab, embed, seq, n_workers)
    out = fn(idx, word_table, pos_table)
    return out.reshape(batch, seq, embed)


# speedup vs baseline: 1.3615x; 1.0279x over previous
import functools

import jax
import jax.numpy as jnp
from jax import lax
from jax.experimental import pallas as pl
from jax.experimental.pallas import tpu as pltpu
from jax.experimental.pallas import tpu_sc as plsc

LANES = 16
CHUNK = 128
NBUF = 4
ET = 8


def _build(batch, seq, vocab, embed, n_workers):
    total_rows = batch * seq
    per_w = total_rows // n_workers
    n_chunks = per_w // CHUNK
    n_groups = n_chunks // NBUF
    blocks_per_s = batch // CHUNK
    n_et = embed // ET
    seg = ET * CHUNK
    mesh = plsc.VectorSubcoreMesh(core_axis_name="c", subcore_axis_name="s")
    num_cores = 2

    @functools.partial(
        pl.kernel,
        mesh=mesh,
        compiler_params=pltpu.CompilerParams(use_tc_tiling_on_sc=False,
                                             needs_layout_passes=False),
        out_type=jax.ShapeDtypeStruct((total_rows * embed,), jnp.float32),
        scratch_types=[
            pltpu.VMEM((n_chunks, CHUNK), jnp.int32),
            pltpu.VMEM((NBUF, CHUNK, embed), jnp.float32),
            pltpu.VMEM((CHUNK * embed,), jnp.float32),
            pltpu.VMEM((CHUNK * embed,), jnp.float32),
            pltpu.VMEM((CHUNK * embed,), jnp.float32),
            pltpu.VMEM((CHUNK * embed,), jnp.float32),
            pltpu.VMEM((seq, embed), jnp.float32),
            pltpu.SemaphoreType.DMA,
            pltpu.SemaphoreType.DMA,
            pltpu.SemaphoreType.DMA,
            pltpu.SemaphoreType.DMA,
            pltpu.SemaphoreType.DMA,
            pltpu.SemaphoreType.DMA,
            pltpu.SemaphoreType.DMA,
            pltpu.SemaphoreType.DMA,
        ],
    )
    def emb_kernel(idx_hbm, table_hbm, pos_hbm, out_hbm,
                   idx_all, rows_v, t0, t1, t2, t3, pos_v, *sems):
        rowst = (t0, t1, t2, t3)
        semg = sems[:NBUF]
        semw = sems[NBUF:]
        wid = lax.axis_index("s") * num_cores + lax.axis_index("c")
        f0 = wid * n_chunks

        pltpu.sync_copy(pos_hbm, pos_v)
        pltpu.sync_copy(idx_hbm.at[pl.ds(f0, n_chunks)], idx_all)

        iota_e = lax.iota(jnp.int32, LANES) * CHUNK

        def fire(c, b):
            pltpu.async_copy(table_hbm.at[idx_all.at[c]], rows_v.at[b],
                             semg[b])

        def drain_g(b):
            pltpu.make_async_copy(table_hbm.at[idx_all.at[0]], rows_v.at[b],
                                  semg[b]).wait()

        def drain_w(b):
            pltpu.make_async_copy(rowst[b],
                                  out_hbm.at[pl.ds(0, CHUNK * embed)],
                                  semw[b]).wait()

        def transpose_add(s, b):
            pos_parts = [pos_v[s, pl.ds(j * LANES, LANES)]
                         for j in range(embed // LANES)]

            def r_body(r, _):
                for j in range(embed // LANES):
                    v = rows_v[b, r, pl.ds(j * LANES, LANES)] + pos_parts[j]
                    idx = iota_e + (j * LANES * CHUNK + r)
                    plsc.store_scatter(rowst[b], [idx], v)
                return _

            lax.fori_loop(0, CHUNK, r_body, None)

        def step(c, b, wait_w, fire_ahead):
            drain_g(b)
            s = c // blocks_per_s
            bt = lax.rem(c, blocks_per_s)
            transpose_add(s, b)
            obase = s * (embed * batch) + bt * (ET * CHUNK)
            for et in range(n_et):
                pltpu.async_copy(
                    rowst[b].at[pl.ds(et * seg, seg)],
                    out_hbm.at[pl.ds(obase + et * (blocks_per_s * seg), seg)],
                    semw[b])
            bf = (b + NBUF - 1) % NBUF
            if wait_w:
                drain_w(bf)
            if fire_ahead:
                fire(c - f0 + NBUF - 1, bf)

        for b in range(NBUF - 1):
            fire(b, b)
        for b in range(NBUF):
            step(f0 + b, b, wait_w=(b > 0), fire_ahead=True)

        def group_body(g, _):
            c0 = f0 + g * NBUF
            for b in range(NBUF):
                step(c0 + b, b, wait_w=True, fire_ahead=True)
            return _

        lax.fori_loop(1, n_groups - 1, group_body, None)

        cL = f0 + (n_groups - 1) * NBUF
        step(cL, 0, wait_w=True, fire_ahead=True)
        for b in range(1, NBUF):
            step(cL + b, b, wait_w=True, fire_ahead=False)
        drain_w(NBUF - 1)

    return emb_kernel


def kernel(inputs, word_table, pos_table):
    batch, seq = inputs.shape
    vocab, embed = word_table.shape
    n_workers = 32

    idx = inputs.T.reshape(batch * seq // CHUNK, CHUNK).astype(jnp.int32)
    fn = _build(batch, seq, vocab, embed, n_workers)
    flat = fn(idx, word_table, pos_table)
    x = flat.reshape(seq, embed // ET, batch // CHUNK, ET, CHUNK)
    return x.transpose(2, 4, 0, 1, 3).reshape(batch, seq, embed)
